# baseline (speedup 1.0000x reference)
import jax
import jax.numpy as jnp
from jax import lax
from jax.experimental import pallas as pl
from jax.experimental.pallas import tpu as pltpu
from jax.experimental.pallas import tpu_sc as plsc

_USER_NUM = 20000
_ITEM_NUM = 30000
_N = _USER_NUM + _ITEM_NUM
_N_PAD = 50048
_E = 800000
_D = 64
_HALF = _D // 2

_NC = 2
_NS = 16

_CHUNK = 256
_ROWS_PER_CHUNK = _CHUNK // 128
_CHUNKS_PER_TILE = 196
_GROUP = 7
_NGROUPS = _CHUNKS_PER_TILE // _GROUP
_IDXROWS_PER_TILE = _CHUNKS_PER_TILE * _ROWS_PER_CHUNK
_E_PAD = _NS * _CHUNK * _CHUNKS_PER_TILE
_IDXROWS = _E_PAD // 128
_NODE_ROWS_PER_TILE = _N_PAD // _NS

_BLK = 2176
_GRID = _N_PAD // _BLK


def _spmm_body(pack_hbm, xlo_hbm, xhi_hbm, zeros_hbm,
               ylo_hbm, yhi_hbm,
               accum, idxg, rows, srows, sem):
    c = lax.axis_index("c")
    s = lax.axis_index("s")

    nr0 = s * _NODE_ROWS_PER_TILE
    pltpu.sync_copy(zeros_hbm.at[pl.ds(nr0, _NODE_ROWS_PER_TILE)],
                    accum.at[pl.ds(nr0, _NODE_ROWS_PER_TILE)])
    plsc.subcore_barrier()

    base_row = s * _IDXROWS_PER_TILE

    def gather_descs(jc, buf, x_hbm):
        b = (jc // _GROUP) % 2
        rl = jc % _GROUP
        return [pltpu.make_async_copy(
                    x_hbm.at[idxg.at[b, 2 * rl + rr, 0]],
                    rows.at[buf, pl.ds(rr * 128, 128)], sem)
                for rr in range(_ROWS_PER_CHUNK)]

    pltpu.sync_copy(pack_hbm.at[pl.ds(base_row, 2 * _GROUP)], idxg.at[0])

    @pl.when(c == 0)
    def _():
        for d in gather_descs(0, 0, xlo_hbm):
            d.start()
        for d in gather_descs(0, 0, xlo_hbm):
            d.wait()

    @pl.when(c == 1)
    def _():
        for d in gather_descs(0, 0, xhi_hbm):
            d.start()
        for d in gather_descs(0, 0, xhi_hbm):
            d.wait()

    def chunk_body(j, carry):
        q = j // _GROUP
        rl = j % _GROUP
        b = q % 2
        p = j % 2

        @pl.when(jnp.logical_and(rl == 0, q < _NGROUPS - 1))
        def _():
            pltpu.sync_copy(
                pack_hbm.at[pl.ds(base_row + (q + 1) * 2 * _GROUP,
                                  2 * _GROUP)],
                idxg.at[1 - b])

        @pl.when(j < _CHUNKS_PER_TILE - 1)
        def _():
            @pl.when(c == 0)
            def _():
                for d in gather_descs(j + 1, 1 - p, xlo_hbm):
                    d.start()

            @pl.when(c == 1)
            def _():
                for d in gather_descs(j + 1, 1 - p, xhi_hbm):
                    d.start()

        for rr in range(_ROWS_PER_CHUNK):
            for gg in range(8):
                v16 = plsc.bitcast(
                    idxg[b, 2 * rl + rr, 2, pl.ds(gg * 16, 16)], jnp.float32)
                for i in range(16):
                    e = rr * 128 + gg * 16 + i
                    v = v16[i]
                    rb = plsc.bitcast(rows[p, e, :], jnp.bfloat16)
                    r0, r1 = plsc.unpack(rb, format=plsc.PackFormat.INTERLEAVED)
                    srows[e, pl.ds(0, 16)] = r0 * v
                    srows[e, pl.ds(16, 16)] = r1 * v

        for rr in range(_ROWS_PER_CHUNK):
            pltpu.sync_copy(srows.at[pl.ds(rr * 128, 128)],
                            accum.at[idxg.at[b, 2 * rl + rr, 1]], add=True)

        @pl.when(j < _CHUNKS_PER_TILE - 1)
        def _():
            @pl.when(c == 0)
            def _():
                for d in gather_descs(j + 1, 1 - p, xlo_hbm):
                    d.wait()

            @pl.when(c == 1)
            def _():
                for d in gather_descs(j + 1, 1 - p, xhi_hbm):
                    d.wait()

        return carry

    lax.fori_loop(0, _CHUNKS_PER_TILE, chunk_body, 0)

    plsc.subcore_barrier()

    @pl.when(c == 0)
    def _():
        pltpu.sync_copy(accum.at[pl.ds(nr0, _NODE_ROWS_PER_TILE)],
                        ylo_hbm.at[pl.ds(nr0, _NODE_ROWS_PER_TILE)])

    @pl.when(c == 1)
    def _():
        pltpu.sync_copy(accum.at[pl.ds(nr0, _NODE_ROWS_PER_TILE)],
                        yhi_hbm.at[pl.ds(nr0, _NODE_ROWS_PER_TILE)])


_spmm = pl.kernel(
    _spmm_body,
    out_type=(jax.ShapeDtypeStruct((_N_PAD, _HALF), jnp.float32),
              jax.ShapeDtypeStruct((_N_PAD, _HALF), jnp.float32)),
    mesh=plsc.VectorSubcoreMesh(core_axis_name="c", subcore_axis_name="s",
                                num_cores=_NC, num_subcores=_NS),
    compiler_params=pltpu.CompilerParams(use_tc_tiling_on_sc=False,
                                     needs_layout_passes=False),
    scratch_types=(
        pltpu.VMEM_SHARED((_N_PAD, _HALF), jnp.float32),
        pltpu.VMEM((2, 2 * _GROUP, 3, 128), jnp.int32),
        pltpu.VMEM((2, _CHUNK, _HALF // 2), jnp.int32),
        pltpu.VMEM((_CHUNK, _HALF), jnp.float32),
        pltpu.SemaphoreType.DMA,
    ),
)


def _pack_bf16_pairs(h):
    au = jax.lax.bitcast_convert_type(
        h[:, :16].astype(jnp.bfloat16), jnp.uint16).astype(jnp.uint32)
    bu = jax.lax.bitcast_convert_type(
        h[:, 16:].astype(jnp.bfloat16), jnp.uint16).astype(jnp.uint32)
    return jax.lax.bitcast_convert_type(au | (bu << 16), jnp.int32)


def _dense_layer_body(w1_ref, w2_ref, xlo_ref, xhi_ref, alo_ref, ahi_ref,
                      aclo_ref, achi_ref,
                      ylo_ref, yhi_ref, oaclo_ref, oachi_ref,
                      yblo_ref, ybhi_ref):
    x = jnp.concatenate([xlo_ref[...], xhi_ref[...]], axis=1)
    a = jnp.concatenate([alo_ref[...], ahi_ref[...]], axis=1)
    y = (jnp.dot(a + x, w1_ref[...], preferred_element_type=jnp.float32)
         + jnp.dot(a * x, w2_ref[...], preferred_element_type=jnp.float32))
    y = jnp.where(y >= 0.0, y, 0.01 * y)
    ylo_ref[...] = y[:, :_HALF]
    yhi_ref[...] = y[:, _HALF:]
    oaclo_ref[...] = aclo_ref[...] + y[:, :_HALF]
    oachi_ref[...] = achi_ref[...] + y[:, _HALF:]
    yblo_ref[...] = _pack_bf16_pairs(y[:, :_HALF])
    ybhi_ref[...] = _pack_bf16_pairs(y[:, _HALF:])


def _dense_final_body(w1_ref, w2_ref, xlo_ref, xhi_ref, alo_ref, ahi_ref,
                      aclo_ref, achi_ref, out_ref):
    x = jnp.concatenate([xlo_ref[...], xhi_ref[...]], axis=1)
    a = jnp.concatenate([alo_ref[...], ahi_ref[...]], axis=1)
    y = (jnp.dot(a + x, w1_ref[...], preferred_element_type=jnp.float32)
         + jnp.dot(a * x, w2_ref[...], preferred_element_type=jnp.float32))
    y = jnp.where(y >= 0.0, y, 0.01 * y)
    acc = jnp.concatenate([aclo_ref[...], achi_ref[...]], axis=1)
    out_ref[...] = (acc + y) * 0.25


_half_spec = pl.BlockSpec((_BLK, _HALF), lambda i: (i, 0))
_q_spec = pl.BlockSpec((_BLK, _HALF // 2), lambda i: (i, 0))
_w_spec = pl.BlockSpec((_D, _D), lambda i: (0, 0))

_dense_layer = pl.pallas_call(
    _dense_layer_body,
    grid=(_GRID,),
    in_specs=[_w_spec, _w_spec] + [_half_spec] * 6,
    out_specs=[_half_spec] * 4 + [_q_spec] * 2,
    out_shape=[jax.ShapeDtypeStruct((_N_PAD, _HALF), jnp.float32)] * 4
    + [jax.ShapeDtypeStruct((_N_PAD, _HALF // 2), jnp.int32)] * 2,
)

_dense_final = pl.pallas_call(
    _dense_final_body,
    grid=(_GRID,),
    in_specs=[_w_spec, _w_spec] + [_half_spec] * 6,
    out_specs=pl.BlockSpec((_BLK, _D), lambda i: (i, 0)),
    out_shape=jax.ShapeDtypeStruct((_N_PAD, _D), jnp.float32),
)


def kernel(user_emb, item_emb, adj_src, adj_dst, adj_val,
           w1_0, w1_1, w1_2, w2_0, w2_1, w2_2):
    ego = jnp.concatenate(
        [user_emb, item_emb,
         jnp.zeros((_N_PAD - _N, _D), jnp.float32)], axis=0)
    xlo = ego[:, :_HALF]
    xhi = ego[:, _HALF:]

    pad = _E_PAD - _E
    src = jnp.concatenate([adj_src.astype(jnp.int32),
                           jnp.zeros((pad,), jnp.int32)]).reshape(_IDXROWS, 128)
    dst = jnp.concatenate([adj_dst.astype(jnp.int32),
                           jnp.zeros((pad,), jnp.int32)]).reshape(_IDXROWS, 128)
    val = jax.lax.bitcast_convert_type(
        jnp.concatenate([adj_val.astype(jnp.float32),
                         jnp.zeros((pad,), jnp.float32)]),
        jnp.int32).reshape(_IDXROWS, 128)
    pack = jnp.stack([src, dst, val], axis=1)
    zeros = jnp.zeros((_N_PAD, _HALF), jnp.float32)

    W1 = [w1_0, w1_1, w1_2]
    W2 = [w2_0, w2_1, w2_2]

    aclo, achi = xlo, xhi

    def _pack_pairs_host(h):
        au = jax.lax.bitcast_convert_type(
            h[:, :16].astype(jnp.bfloat16), jnp.uint16).astype(jnp.uint32)
        bu = jax.lax.bitcast_convert_type(
            h[:, 16:].astype(jnp.bfloat16), jnp.uint16).astype(jnp.uint32)
        return jax.lax.bitcast_convert_type(au | (bu << 16), jnp.int32)

    xblo = _pack_pairs_host(xlo)
    xbhi = _pack_pairs_host(xhi)
    mean = None
    for k in range(3):
        alo, ahi = _spmm(pack, xblo, xbhi, zeros)
        if k < 2:
            xlo, xhi, aclo, achi, xblo, xbhi = _dense_layer(
                W1[k], W2[k], xlo, xhi, alo, ahi, aclo, achi)
        else:
            mean = _dense_final(
                W1[k], W2[k], xlo, xhi, alo, ahi, aclo, achi)
    return mean[:_USER_NUM], mean[_USER_NUM:_N]


# speedup vs baseline: 2.0345x; 2.0345x over previous
import jax
import jax.numpy as jnp
from jax import lax
from jax.experimental import pallas as pl
from jax.experimental.pallas import tpu as pltpu
from jax.experimental.pallas import tpu_sc as plsc

_USER_NUM = 20000
_ITEM_NUM = 30000
_N = _USER_NUM + _ITEM_NUM
_N_PAD = 50048
_E = 800000
_D = 64
_HALF = _D // 2

_NC = 2
_NS = 16

_CHUNK = 256
_ROWS_PER_CHUNK = _CHUNK // 128
_CHUNKS_PER_TILE = 196
_GROUP = 7
_NGROUPS = _CHUNKS_PER_TILE // _GROUP
_IDXROWS_PER_TILE = _CHUNKS_PER_TILE * _ROWS_PER_CHUNK
_E_PAD = _NS * _CHUNK * _CHUNKS_PER_TILE
_IDXROWS = _E_PAD // 128
_NODE_ROWS_PER_TILE = _N_PAD // _NS

_BLK = 3128
_GRID = _N_PAD // _BLK


def _spmm_body(pack_hbm, xlo_hbm, xhi_hbm, zeros_hbm,
               ylo_hbm, yhi_hbm,
               accum, idxg, rows, sem):
    c = lax.axis_index("c")
    s = lax.axis_index("s")

    nr0 = s * _NODE_ROWS_PER_TILE
    pltpu.sync_copy(zeros_hbm.at[pl.ds(nr0, _NODE_ROWS_PER_TILE)],
                    accum.at[pl.ds(nr0, _NODE_ROWS_PER_TILE)])
    plsc.subcore_barrier()

    base_row = s * _IDXROWS_PER_TILE

    def gather_descs(jc, buf, x_hbm):
        b = (jc // _GROUP) % 2
        rl = jc % _GROUP
        return [pltpu.make_async_copy(
                    x_hbm.at[idxg.at[b, 2 * rl + rr, 0]],
                    rows.at[buf, pl.ds(rr * 128, 128)], sem)
                for rr in range(_ROWS_PER_CHUNK)]

    pltpu.sync_copy(pack_hbm.at[pl.ds(base_row, 2 * _GROUP)], idxg.at[0])

    @pl.when(c == 0)
    def _():
        for d in gather_descs(0, 0, xlo_hbm):
            d.start()
        for d in gather_descs(0, 0, xlo_hbm):
            d.wait()

    @pl.when(c == 1)
    def _():
        for d in gather_descs(0, 0, xhi_hbm):
            d.start()
        for d in gather_descs(0, 0, xhi_hbm):
            d.wait()

    def chunk_body(j, carry):
        q = j // _GROUP
        rl = j % _GROUP
        b = q % 2
        p = j % 2

        @pl.when(jnp.logical_and(rl == 0, q < _NGROUPS - 1))
        def _():
            pltpu.sync_copy(
                pack_hbm.at[pl.ds(base_row + (q + 1) * 2 * _GROUP,
                                  2 * _GROUP)],
                idxg.at[1 - b])

        @pl.when(j < _CHUNKS_PER_TILE - 1)
        def _():
            @pl.when(c == 0)
            def _():
                for d in gather_descs(j + 1, 1 - p, xlo_hbm):
                    d.start()

            @pl.when(c == 1)
            def _():
                for d in gather_descs(j + 1, 1 - p, xhi_hbm):
                    d.start()

        for rr in range(_ROWS_PER_CHUNK):
            for gg in range(8):
                v16 = plsc.bitcast(
                    idxg[b, 2 * rl + rr, 2, pl.ds(gg * 16, 16)], jnp.float32)
                for i in range(16):
                    e = rr * 128 + gg * 16 + i
                    v = v16[i]
                    rows[p, e, pl.ds(0, 16)] = rows[p, e, pl.ds(0, 16)] * v
                    rows[p, e, pl.ds(16, 16)] = rows[p, e, pl.ds(16, 16)] * v

        for rr in range(_ROWS_PER_CHUNK):
            pltpu.sync_copy(rows.at[p, pl.ds(rr * 128, 128)],
                            accum.at[idxg.at[b, 2 * rl + rr, 1]], add=True)

        @pl.when(j < _CHUNKS_PER_TILE - 1)
        def _():
            @pl.when(c == 0)
            def _():
                for d in gather_descs(j + 1, 1 - p, xlo_hbm):
                    d.wait()

            @pl.when(c == 1)
            def _():
                for d in gather_descs(j + 1, 1 - p, xhi_hbm):
                    d.wait()

        return carry

    lax.fori_loop(0, _CHUNKS_PER_TILE, chunk_body, 0)

    plsc.subcore_barrier()

    @pl.when(c == 0)
    def _():
        pltpu.sync_copy(accum.at[pl.ds(nr0, _NODE_ROWS_PER_TILE)],
                        ylo_hbm.at[pl.ds(nr0, _NODE_ROWS_PER_TILE)])

    @pl.when(c == 1)
    def _():
        pltpu.sync_copy(accum.at[pl.ds(nr0, _NODE_ROWS_PER_TILE)],
                        yhi_hbm.at[pl.ds(nr0, _NODE_ROWS_PER_TILE)])


_spmm = pl.kernel(
    _spmm_body,
    out_type=(jax.ShapeDtypeStruct((_N_PAD, _HALF), jnp.float32),
              jax.ShapeDtypeStruct((_N_PAD, _HALF), jnp.float32)),
    mesh=plsc.VectorSubcoreMesh(core_axis_name="c", subcore_axis_name="s",
                                num_cores=_NC, num_subcores=_NS),
    compiler_params=pltpu.CompilerParams(use_tc_tiling_on_sc=False,
                                     needs_layout_passes=False),
    scratch_types=(
        pltpu.VMEM_SHARED((_N_PAD, _HALF), jnp.float32),
        pltpu.VMEM((2, 2 * _GROUP, 3, 128), jnp.int32),
        pltpu.VMEM((2, _CHUNK, _HALF), jnp.float32),
        pltpu.SemaphoreType.DMA,
    ),
)


def _dense_layer_body(w1_ref, w2_ref, xlo_ref, xhi_ref, alo_ref, ahi_ref,
                      aclo_ref, achi_ref,
                      ylo_ref, yhi_ref, oaclo_ref, oachi_ref):
    x = jnp.concatenate([xlo_ref[...], xhi_ref[...]], axis=1)
    a = jnp.concatenate([alo_ref[...], ahi_ref[...]], axis=1)
    y = (jnp.dot(a + x, w1_ref[...], preferred_element_type=jnp.float32)
         + jnp.dot(a * x, w2_ref[...], preferred_element_type=jnp.float32))
    y = jnp.where(y >= 0.0, y, 0.01 * y)
    ylo_ref[...] = y[:, :_HALF]
    yhi_ref[...] = y[:, _HALF:]
    oaclo_ref[...] = aclo_ref[...] + y[:, :_HALF]
    oachi_ref[...] = achi_ref[...] + y[:, _HALF:]


def _dense_final_body(w1_ref, w2_ref, xlo_ref, xhi_ref, alo_ref, ahi_ref,
                      aclo_ref, achi_ref, out_ref):
    x = jnp.concatenate([xlo_ref[...], xhi_ref[...]], axis=1)
    a = jnp.concatenate([alo_ref[...], ahi_ref[...]], axis=1)
    y = (jnp.dot(a + x, w1_ref[...], preferred_element_type=jnp.float32)
         + jnp.dot(a * x, w2_ref[...], preferred_element_type=jnp.float32))
    y = jnp.where(y >= 0.0, y, 0.01 * y)
    acc = jnp.concatenate([aclo_ref[...], achi_ref[...]], axis=1)
    out_ref[...] = (acc + y) * 0.25


_half_spec = pl.BlockSpec((_BLK, _HALF), lambda i: (i, 0))
_w_spec = pl.BlockSpec((_D, _D), lambda i: (0, 0))

_dense_layer = pl.pallas_call(
    _dense_layer_body,
    grid=(_GRID,),
    in_specs=[_w_spec, _w_spec] + [_half_spec] * 6,
    out_specs=[_half_spec] * 4,
    out_shape=[jax.ShapeDtypeStruct((_N_PAD, _HALF), jnp.float32)] * 4,
)

_dense_final = pl.pallas_call(
    _dense_final_body,
    grid=(_GRID,),
    in_specs=[_w_spec, _w_spec] + [_half_spec] * 6,
    out_specs=pl.BlockSpec((_BLK, _D), lambda i: (i, 0)),
    out_shape=jax.ShapeDtypeStruct((_N_PAD, _D), jnp.float32),
)


def kernel(user_emb, item_emb, adj_src, adj_dst, adj_val,
           w1_0, w1_1, w1_2, w2_0, w2_1, w2_2):
    ego = jnp.concatenate(
        [user_emb, item_emb,
         jnp.zeros((_N_PAD - _N, _D), jnp.float32)], axis=0)
    xlo = ego[:, :_HALF]
    xhi = ego[:, _HALF:]

    pad = _E_PAD - _E
    src = jnp.concatenate([adj_src.astype(jnp.int32),
                           jnp.zeros((pad,), jnp.int32)]).reshape(_IDXROWS, 128)
    dst = jnp.concatenate([adj_dst.astype(jnp.int32),
                           jnp.zeros((pad,), jnp.int32)]).reshape(_IDXROWS, 128)
    val = jax.lax.bitcast_convert_type(
        jnp.concatenate([adj_val.astype(jnp.float32),
                         jnp.zeros((pad,), jnp.float32)]),
        jnp.int32).reshape(_IDXROWS, 128)
    pack = jnp.stack([src, dst, val], axis=1)
    zeros = jnp.zeros((_N_PAD, _HALF), jnp.float32)

    W1 = [w1_0, w1_1, w1_2]
    W2 = [w2_0, w2_1, w2_2]

    aclo, achi = xlo, xhi
    mean = None
    for k in range(3):
        alo, ahi = _spmm(pack, xlo, xhi, zeros)
        if k < 2:
            xlo, xhi, aclo, achi = _dense_layer(
                W1[k], W2[k], xlo, xhi, alo, ahi, aclo, achi)
        else:
            mean = _dense_final(
                W1[k], W2[k], xlo, xhi, alo, ahi, aclo, achi)
    return mean[:_USER_NUM], mean[_USER_NUM:_N]
